# baseline (speedup 1.0000x reference)
import functools

import jax
import jax.numpy as jnp
from jax.experimental import pallas as pl
from jax.experimental.pallas import tpu as pltpu

EMBED_DIM = 512
N_OUT = 256


def _mlp_block(p1_ref, p2_ref, w1_ref, b1_ref, w2_ref, b2_ref, out_ref):
    bf16 = jnp.bfloat16
    p1 = p1_ref[...].astype(bf16)
    p2 = p2_ref[...].astype(bf16)
    w1 = w1_ref[...].astype(bf16)
    b1 = b1_ref[...].astype(bf16)
    h = p1 * w1[0:1, :] + p2 * w1[1:2, :] + b1
    h = h * jax.nn.sigmoid(h)
    out_ref[...] = (
        jnp.dot(h, w2_ref[...].astype(bf16), preferred_element_type=jnp.float32)
        + b2_ref[...]
    )


@functools.partial(jax.jit, static_argnames=())
def kernel(pos1, pos2, W1, b1, W2, b2):
    B, S = pos1.shape
    N = B * S
    T = 2048
    grid = (N // T,)

    p1 = pos1.reshape(N, 1)
    p2 = pos2.reshape(N, 1)
    b1r = b1.reshape(1, EMBED_DIM)
    b2r = b2.reshape(1, N_OUT)

    tok_spec = pl.BlockSpec((T, 1), lambda i: (i, 0))
    full = lambda shape: pl.BlockSpec(shape, lambda i: (0, 0))

    out = pl.pallas_call(
        _mlp_block,
        grid=grid,
        in_specs=[
            tok_spec,
            tok_spec,
            full((2, EMBED_DIM)),
            full((1, EMBED_DIM)),
            full((EMBED_DIM, N_OUT)),
            full((1, N_OUT)),
        ],
        out_specs=pl.BlockSpec((T, N_OUT), lambda i: (i, 0)),
        out_shape=jax.ShapeDtypeStruct((N, N_OUT), jnp.float32),
        compiler_params=pltpu.CompilerParams(
            dimension_semantics=("parallel",),
        ),
    )(p1, p2, W1, b1r, W2, b2r)
    return out.reshape(B, S, N_OUT)


# speedup vs baseline: 1.4765x; 1.4765x over previous
import functools

import jax
import jax.numpy as jnp
from jax.experimental import pallas as pl
from jax.experimental.pallas import tpu as pltpu

EMBED_DIM = 512
N_OUT = 256


def _mlp_block(x_ref, w1_ref, b1_ref, w2_ref, b2_ref, out_ref):
    bf16 = jnp.bfloat16
    xb = x_ref[...].astype(bf16)
    w1 = w1_ref[...].astype(bf16)
    h = jax.lax.dot_general(
        xb, w1, (((0,), (0,)), ((), ())),
        preferred_element_type=jnp.float32,
    ) + b1_ref[...]
    h = h.astype(bf16)
    h = h * jax.nn.sigmoid(h)
    out_ref[...] = (
        jnp.dot(h, w2_ref[...].astype(bf16), preferred_element_type=jnp.float32)
        + b2_ref[...]
    )


@functools.partial(jax.jit, static_argnames=())
def kernel(pos1, pos2, W1, b1, W2, b2):
    B, S = pos1.shape
    N = B * S
    T = 2048
    grid = (N // T,)

    x2 = jnp.stack((pos1.reshape(N), pos2.reshape(N)), axis=0)
    b1r = b1.reshape(1, EMBED_DIM)
    b2r = b2.reshape(1, N_OUT)

    full = lambda shape: pl.BlockSpec(shape, lambda i: (0, 0))

    out = pl.pallas_call(
        _mlp_block,
        grid=grid,
        in_specs=[
            pl.BlockSpec((2, T), lambda i: (0, i)),
            full((2, EMBED_DIM)),
            full((1, EMBED_DIM)),
            full((EMBED_DIM, N_OUT)),
            full((1, N_OUT)),
        ],
        out_specs=pl.BlockSpec((T, N_OUT), lambda i: (i, 0)),
        out_shape=jax.ShapeDtypeStruct((N, N_OUT), jnp.float32),
        compiler_params=pltpu.CompilerParams(
            dimension_semantics=("parallel",),
        ),
    )(x2, W1, b1r, W2, b2r)
    return out.reshape(B, S, N_OUT)
